# Initial kernel scaffold; baseline (speedup 1.0000x reference)
import jax
import jax.numpy as jnp
from jax.experimental import pallas as pl


def kernel(docs, table, W, b):
    raise NotImplementedError("write your pallas kernel here")


# speedup vs baseline: 25.9894x; 25.9894x over previous
import jax
import jax.numpy as jnp
from jax import lax
from jax.experimental import pallas as pl
from jax.experimental.pallas import tpu as pltpu
from jax.experimental.pallas import tpu_sc as plsc

VOCAB = 1000
EMB = 300
NCLS = 20
B = 4096
L = 50

NC, NS = 2, 16
NW = NC * NS
DOCS_PER_W = B // NW
GROUPS = DOCS_PER_W // 16


def _tc_project(table_ref, w_ref, b_ref, m_ref, bias_ref):
    m_ref[...] = lax.dot_general(
        table_ref[...], w_ref[...],
        (((1,), (1,)), ((), ())),
        preferred_element_type=jnp.float32,
    ) * (1.0 / L)
    bias_ref[...] = jnp.broadcast_to(b_ref[...], (NCLS, 16))


def _sc_pool(m_hbm, bias_hbm, docs_hbm, out_hbm, m_v, bias_v, docs_v, out_v):
    cid = lax.axis_index("c")
    sid = lax.axis_index("s")
    wid = sid * NC + cid
    pltpu.sync_copy(m_hbm, m_v)
    pltpu.sync_copy(bias_hbm, bias_v)
    pltpu.sync_copy(
        docs_hbm.at[pl.ds(wid * (DOCS_PER_W * L), DOCS_PER_W * L)], docs_v)
    lane = lax.iota(jnp.int32, 16)
    for g in range(GROUPS):
        doc = lane + g * 16
        tok_base = doc * L
        acc0 = tuple(bias_v[c, :] for c in range(NCLS))

        def step(l, accs, tok_base=tok_base):
            tok = plsc.load_gather(docs_v, [tok_base + l])
            mbase = tok * NCLS
            return tuple(accs[c] + plsc.load_gather(m_v, [mbase + c])
                         for c in range(NCLS))

        accs = lax.fori_loop(0, L, step, acc0)
        out_base = doc * NCLS
        for c in range(NCLS):
            plsc.store_scatter(out_v, [out_base + c], accs[c])
    pltpu.sync_copy(
        out_v,
        out_hbm.at[pl.ds(wid * (DOCS_PER_W * NCLS), DOCS_PER_W * NCLS)])


def kernel(docs, table, W, b):
    m, bias_b = pl.pallas_call(
        _tc_project,
        out_shape=(
            jax.ShapeDtypeStruct((VOCAB, NCLS), jnp.float32),
            jax.ShapeDtypeStruct((NCLS, 16), jnp.float32),
        ),
    )(table, W, b.reshape(NCLS, 1))

    mesh = plsc.VectorSubcoreMesh(core_axis_name="c", subcore_axis_name="s",
                                  num_cores=NC, num_subcores=NS)
    sc = pl.kernel(
        _sc_pool,
        out_type=jax.ShapeDtypeStruct((B * NCLS,), jnp.float32),
        mesh=mesh,
        compiler_params=pltpu.CompilerParams(needs_layout_passes=False),
        scratch_types=[
            pltpu.VMEM((VOCAB * NCLS,), jnp.float32),
            pltpu.VMEM((NCLS, 16), jnp.float32),
            pltpu.VMEM((DOCS_PER_W * L,), jnp.int32),
            pltpu.VMEM((DOCS_PER_W * NCLS,), jnp.float32),
        ],
    )
    out_flat = sc(m.reshape(-1), bias_b, docs.astype(jnp.int32).reshape(-1))
    return out_flat.reshape(B, NCLS)
